# baseline (speedup 1.0000x reference)
import functools
import jax
import jax.numpy as jnp
from jax import lax
from jax.experimental import pallas as pl
from jax.experimental.pallas import tpu as pltpu
from jax.experimental.pallas import tpu_sc as plsc

NC = 2
NS = 16
PADR = 8
KCH = 13


def _p1_body(inp_ref, met_ref, v_ref, *, BP):
    k = pl.program_id(1)
    e = jnp.exp(met_ref[0, 0])
    x = inp_ref[0, 0] * e[None]

    @pl.when(k < 12)
    def _():
        v_ref[0, 0] = x.T

    @pl.when(k == 12)
    def _():
        v_ref[0, 0] = jnp.concatenate(
            [e[:, None], jnp.zeros((BP, 7), jnp.float32)], axis=1)


def _p2_body(flow_ref, idx_ref, w_ref, *, H, W, BH):
    i = pl.program_id(1)
    fx = flow_ref[0, 0]
    fy = flow_ref[0, 1]
    yb = i * BH
    yi = lax.broadcasted_iota(jnp.int32, (BH, W), 0) + yb
    xi = lax.broadcasted_iota(jnp.int32, (BH, W), 1)
    yf = yi.astype(jnp.float32)
    xf = xi.astype(jnp.float32)
    Xp = xf + fx
    Yp = yf + fy
    x0 = jnp.floor(Xp)
    y0 = jnp.floor(Yp)
    x1 = x0 + 1.0
    y1 = y0 + 1.0
    w_nw = (x1 - Xp) * (y1 - Yp)
    w_ne = (Xp - x0) * (y1 - Yp)
    w_sw = (x1 - Xp) * (Yp - y0)
    w_se = (Xp - x0) * (Yp - y0)
    x0i = x0.astype(jnp.int32)
    y0i = y0.astype(jnp.int32)
    x1i = x0i + 1
    y1i = y0i + 1
    mx0 = ((x0i >= 0) & (x0i < W)).astype(jnp.float32)
    mx1 = ((x1i >= 0) & (x1i < W)).astype(jnp.float32)
    my0 = (y0i >= 0) & (y0i < H)
    my1 = (y1i >= 0) & (y1i < H)
    my0f = my0.astype(jnp.float32)
    my1f = my1.astype(jnp.float32)

    p = yi * W + xi
    dump = (p & 3) + 1

    act_n = my0 & (x0i >= -1) & (x0i <= W - 1)
    act_s = my1 & (x0i >= -1) & (x0i <= W - 1)
    idx_n = jnp.where(act_n, y0i * W + x0i + PADR, dump)
    idx_s = jnp.where(act_s, y1i * W + x0i + PADR, dump)

    idx_ref[0, 0] = idx_n
    idx_ref[0, 1] = idx_n + 1
    idx_ref[0, 2] = idx_s
    idx_ref[0, 3] = idx_s + 1
    w_ref[0, 0] = w_nw * (my0f * mx0)
    w_ref[0, 1] = w_ne * (my0f * mx1)
    w_ref[0, 2] = w_sw * (my1f * mx0)
    w_ref[0, 3] = w_se * (my1f * mx1)


def _make_sc_scatter(HW, G):
    R = HW + 2 * PADR
    PPT = HW // NS
    NGRP = PPT // G
    ZR = 1024
    ZREM = R // NS - (R // NS // ZR) * ZR
    mesh = plsc.VectorSubcoreMesh(core_axis_name="c", subcore_axis_name="s")

    def body(v_hbm, idx_hbm, w_hbm, z_hbm, out_hbm,
             acc, zbuf, vbuf, wbn, wbs, ibuf, sb0, sb1, sb2, sb3):
        c = lax.axis_index("c")
        s = lax.axis_index("s")
        pltpu.sync_copy(z_hbm, zbuf)

        lanes = lax.iota(jnp.int32, 16)
        wpat = lanes >> 3
        cpat = lanes & 7

        @pl.loop(0, KCH)
        def _pass(k):
            pid = c * KCH + k
            zrows = R // NS
            z0 = s * zrows
            for j in range(zrows // ZR):
                pltpu.sync_copy(zbuf, acc.at[pl.ds(z0 + j * ZR, ZR), :])
            if ZREM:
                pltpu.sync_copy(zbuf.at[pl.ds(0, ZREM), :],
                                acc.at[pl.ds(z0 + (zrows // ZR) * ZR, ZREM), :])
            plsc.subcore_barrier()

            base_p = s * PPT

            @pl.loop(0, NGRP)
            def _group(g):
                p0 = base_p + g * G
                pltpu.sync_copy(v_hbm.at[pid, pl.ds(p0 * 8, G * 8)], vbuf)
                pltpu.sync_copy(w_hbm.at[c, 0, 0, pl.ds(p0, G)], wbn.at[pl.ds(0, G)])
                pltpu.sync_copy(w_hbm.at[c, 0, 1, pl.ds(p0, G)], wbn.at[pl.ds(G, G)])
                pltpu.sync_copy(w_hbm.at[c, 1, 0, pl.ds(p0, G)], wbs.at[pl.ds(0, G)])
                pltpu.sync_copy(w_hbm.at[c, 1, 1, pl.ds(p0, G)], wbs.at[pl.ds(G, G)])
                pltpu.sync_copy(idx_hbm.at[c, :, pl.ds(s * (PPT // 128) + g * (G // 128), G // 128), :],
                                ibuf)

                @pl.loop(0, G // 2)
                def _pair(q):
                    q2 = q * 2
                    v2 = vbuf[pl.ds(q2 * 8, 16)]
                    wi0 = q2 + wpat
                    wn0 = plsc.load_gather(wbn, [wi0])
                    wn1 = plsc.load_gather(wbn, [wi0 + G])
                    ws0 = plsc.load_gather(wbs, [wi0])
                    ws1 = plsc.load_gather(wbs, [wi0 + G])
                    ridx = q2 + wpat
                    plsc.store_scatter(sb0, [ridx, cpat], v2 * wn0)
                    plsc.store_scatter(sb1, [ridx, cpat], v2 * wn1)
                    plsc.store_scatter(sb2, [ridx, cpat], v2 * ws0)
                    plsc.store_scatter(sb3, [ridx, cpat], v2 * ws1)

                for cell, sb in enumerate((sb0, sb1, sb2, sb3)):
                    for j in range(G // 128):
                        pltpu.sync_copy(sb.at[pl.ds(j * 128, 128), :],
                                        acc.at[ibuf.at[cell, j]], add=True)

            plsc.subcore_barrier()
            orow = s * PPT
            pltpu.sync_copy(acc.at[pl.ds(PADR + orow, PPT), :],
                            out_hbm.at[pid, pl.ds(orow, PPT), :])
            plsc.subcore_barrier()

    return pl.kernel(
        body,
        out_type=jax.ShapeDtypeStruct((NC * KCH, HW, 8), jnp.float32),
        mesh=mesh,
        compiler_params=pltpu.CompilerParams(
            needs_layout_passes=False, use_tc_tiling_on_sc=False),
        scratch_types=[
            pltpu.VMEM_SHARED((R, 8), jnp.float32),
            pltpu.VMEM((1024, 8), jnp.float32),
            pltpu.VMEM((G * 8,), jnp.float32),
            pltpu.VMEM((2 * G,), jnp.float32),
            pltpu.VMEM((2 * G,), jnp.float32),
            pltpu.VMEM((4, G // 128, 128), jnp.int32),
            pltpu.VMEM((G, 8), jnp.float32),
            pltpu.VMEM((G, 8), jnp.float32),
            pltpu.VMEM((G, 8), jnp.float32),
            pltpu.VMEM((G, 8), jnp.float32),
        ],
    )


def _p3_body(num_ref, nrm_ref, out_ref):
    nrm = nrm_ref[0, 0, :, 0:1]
    x = num_ref[0, 0] / (nrm + 1e-22)
    out_ref[0, 0] = x.T


def kernel(tenInput, tenFlow, tenMetric):
    N, C, H, W = tenInput.shape
    HW = H * W
    BP = 512
    BH = 8
    G = 512

    inp = tenInput.reshape(N, 12, 8, HW)
    met = tenMetric.reshape(N, 1, HW)

    v4 = pl.pallas_call(
        functools.partial(_p1_body, BP=BP),
        grid=(N, KCH, HW // BP),
        in_specs=[
            pl.BlockSpec((1, 1, 8, BP),
                         lambda n, k, j: (n, jnp.minimum(k, 11), 0, j)),
            pl.BlockSpec((1, 1, BP), lambda n, k, j: (n, 0, j)),
        ],
        out_specs=pl.BlockSpec((1, 1, BP, 8), lambda n, k, j: (n, k, j, 0)),
        out_shape=jax.ShapeDtypeStruct((N, KCH, HW, 8), jnp.float32),
    )(inp, met)

    idx, wpl = pl.pallas_call(
        functools.partial(_p2_body, H=H, W=W, BH=BH),
        grid=(N, H // BH),
        in_specs=[pl.BlockSpec((1, 2, BH, W), lambda n, i: (n, 0, i, 0))],
        out_specs=[
            pl.BlockSpec((1, 4, BH, W), lambda n, i: (n, 0, i, 0)),
            pl.BlockSpec((1, 4, BH, W), lambda n, i: (n, 0, i, 0)),
        ],
        out_shape=[
            jax.ShapeDtypeStruct((N, 4, H, W), jnp.int32),
            jax.ShapeDtypeStruct((N, 4, H, W), jnp.float32),
        ],
    )(tenFlow)

    v26 = v4.reshape(N * KCH, HW * 8)

    idx_r = idx.reshape(N, 4, HW // 128, 128)
    wpl_r = wpl.reshape(N, 2, 2, HW)
    zeros = jnp.zeros((1024, 8), jnp.float32)

    sc = _make_sc_scatter(HW, G)
    out26 = sc(v26, idx_r, wpl_r, zeros)

    o4 = out26.reshape(N, KCH, HW, 8)
    d = pl.pallas_call(
        _p3_body,
        grid=(N, 12, HW // BP),
        in_specs=[
            pl.BlockSpec((1, 1, BP, 8), lambda n, k, j: (n, k, j, 0)),
            pl.BlockSpec((1, 1, BP, 8), lambda n, k, j: (n, 12, j, 0)),
        ],
        out_specs=pl.BlockSpec((1, 1, 8, BP), lambda n, k, j: (n, k, 0, j)),
        out_shape=jax.ShapeDtypeStruct((N, 12, 8, HW), jnp.float32),
    )(o4, o4)

    return d.reshape(N, C, H, W)


# speedup vs baseline: 2.3467x; 2.3467x over previous
import functools
import jax
import jax.numpy as jnp
from jax import lax
from jax.experimental import pallas as pl
from jax.experimental.pallas import tpu as pltpu
from jax.experimental.pallas import tpu_sc as plsc

NC = 2
NS = 16
PADR = 8
KCH = 13


def _p1_body(inp_ref, met_ref, v0_ref, e_ref):
    e = jnp.exp(met_ref[...])
    v0_ref[...] = inp_ref[...] * e
    e_ref[...] = e


def _p2_body(flow_ref, idx_ref, w_ref, *, H, W, BH):
    i = pl.program_id(1)
    fx = flow_ref[0, 0]
    fy = flow_ref[0, 1]
    yb = i * BH
    yi = lax.broadcasted_iota(jnp.int32, (BH, W), 0) + yb
    xi = lax.broadcasted_iota(jnp.int32, (BH, W), 1)
    yf = yi.astype(jnp.float32)
    xf = xi.astype(jnp.float32)
    Xp = xf + fx
    Yp = yf + fy
    x0 = jnp.floor(Xp)
    y0 = jnp.floor(Yp)
    x1 = x0 + 1.0
    y1 = y0 + 1.0
    w_nw = (x1 - Xp) * (y1 - Yp)
    w_ne = (Xp - x0) * (y1 - Yp)
    w_sw = (x1 - Xp) * (Yp - y0)
    w_se = (Xp - x0) * (Yp - y0)
    x0i = x0.astype(jnp.int32)
    y0i = y0.astype(jnp.int32)
    x1i = x0i + 1
    y1i = y0i + 1
    mx0 = ((x0i >= 0) & (x0i < W)).astype(jnp.float32)
    mx1 = ((x1i >= 0) & (x1i < W)).astype(jnp.float32)
    my0 = (y0i >= 0) & (y0i < H)
    my1 = (y1i >= 0) & (y1i < H)
    my0f = my0.astype(jnp.float32)
    my1f = my1.astype(jnp.float32)

    p = yi * W + xi
    dump = (p & 3) + 1

    act_n = my0 & (x0i >= -1) & (x0i <= W - 1)
    act_s = my1 & (x0i >= -1) & (x0i <= W - 1)
    idx_n = jnp.where(act_n, y0i * W + x0i + PADR, dump)
    idx_s = jnp.where(act_s, y1i * W + x0i + PADR, dump)

    idx_ref[0, 0] = idx_n
    idx_ref[0, 1] = idx_n + 1
    idx_ref[0, 2] = idx_s
    idx_ref[0, 3] = idx_s + 1
    w_ref[0, 0] = w_nw * (my0f * mx0)
    w_ref[0, 1] = w_ne * (my0f * mx1)
    w_ref[0, 2] = w_sw * (my1f * mx0)
    w_ref[0, 3] = w_se * (my1f * mx1)


def _make_sc_scatter(HW, G):
    R = HW + 2 * PADR
    PPT = HW // NS
    NGRP = PPT // G
    NSUB = G // 128
    ZR = 512
    ZN = (R // NS) // ZR
    ZREM = R // NS - ZN * ZR
    mesh = plsc.VectorSubcoreMesh(core_axis_name="c", subcore_axis_name="s")

    def body(v_hbm, idx_hbm, w_hbm, z_hbm, out_hbm,
             acc, zbuf, vbuf, wbn, wbs, ibuf, sb,
             sem_in0, sem_in1, sem_sc0, sem_sc1, sem_z):
        c = lax.axis_index("c")
        s = lax.axis_index("s")
        sem_in = (sem_in0, sem_in1)
        sem_sc = (sem_sc0, sem_sc1)
        pltpu.sync_copy(z_hbm, zbuf)

        lanes = lax.iota(jnp.int32, 16)
        wpat = lanes >> 3
        cpat = lanes & 7
        base_p = s * PPT
        ib128 = s * (PPT // 128)

        @pl.loop(0, KCH)
        def _pass(k):
            pid = c * KCH + k

            z0 = s * (R // NS)
            for j in range(ZN):
                pltpu.async_copy(zbuf, acc.at[pl.ds(z0 + j * ZR, ZR), :], sem_z)
            if ZREM:
                pltpu.async_copy(zbuf.at[pl.ds(0, ZREM), :],
                                 acc.at[pl.ds(z0 + ZN * ZR, ZREM), :], sem_z)
            for j in range(ZN):
                pltpu.make_async_copy(zbuf, acc.at[pl.ds(z0 + j * ZR, ZR), :], sem_z).wait()
            if ZREM:
                pltpu.make_async_copy(zbuf.at[pl.ds(0, ZREM), :],
                                      acc.at[pl.ds(z0 + ZN * ZR, ZREM), :], sem_z).wait()
            plsc.subcore_barrier()

            def in_copies(g, st):
                p0 = base_p + g * G
                return (
                    (v_hbm.at[pid, pl.ds(p0 * 8, G * 8)], vbuf.at[st]),
                    (w_hbm.at[c, 0, 0, pl.ds(p0, G)], wbn.at[st, pl.ds(0, G)]),
                    (w_hbm.at[c, 0, 1, pl.ds(p0, G)], wbn.at[st, pl.ds(G, G)]),
                    (w_hbm.at[c, 1, 0, pl.ds(p0, G)], wbs.at[st, pl.ds(0, G)]),
                    (w_hbm.at[c, 1, 1, pl.ds(p0, G)], wbs.at[st, pl.ds(G, G)]),
                )

            def fire_inputs(g, st):
                for src, dst in in_copies(g, st):
                    pltpu.async_copy(src, dst, sem_in[st])

            def wait_inputs(g, st):
                for src, dst in in_copies(g, st):
                    pltpu.make_async_copy(src, dst, sem_in[st]).wait()

            def sc_copies(st):
                return tuple(
                    (sb.at[st, cell, pl.ds(j * 128, 128), :],
                     acc.at[ibuf.at[st, cell, j]])
                    for cell in range(4) for j in range(NSUB))

            def fire_scatters(st):
                for src, dst in sc_copies(st):
                    pltpu.async_copy(src, dst, sem_sc[st], add=True)

            def drain_scatters(st):
                for src, dst in sc_copies(st):
                    pltpu.make_async_copy(src, dst, sem_sc[st]).wait()

            fire_inputs(0, 0)
            fire_inputs(1, 1)

            @pl.loop(0, NGRP // 2)
            def _gg(gg):
                for st in (0, 1):
                    g = gg * 2 + st

                    @pl.when(gg > 0)
                    def _():
                        drain_scatters(st)

                    pltpu.sync_copy(
                        idx_hbm.at[c, :, pl.ds(ib128 + g * NSUB, NSUB), :],
                        ibuf.at[st])
                    wait_inputs(g, st)

                    @pl.loop(0, G // 2)
                    def _pair(q):
                        q2 = q * 2
                        v2 = vbuf.at[st][pl.ds(q2 * 8, 16)]
                        wi0 = q2 + wpat
                        wn0 = plsc.load_gather(wbn.at[st], [wi0])
                        wn1 = plsc.load_gather(wbn.at[st], [wi0 + G])
                        ws0 = plsc.load_gather(wbs.at[st], [wi0])
                        ws1 = plsc.load_gather(wbs.at[st], [wi0 + G])
                        ridx = q2 + wpat
                        plsc.store_scatter(sb.at[st, 0], [ridx, cpat], v2 * wn0)
                        plsc.store_scatter(sb.at[st, 1], [ridx, cpat], v2 * wn1)
                        plsc.store_scatter(sb.at[st, 2], [ridx, cpat], v2 * ws0)
                        plsc.store_scatter(sb.at[st, 3], [ridx, cpat], v2 * ws1)

                    @pl.when(gg < NGRP // 2 - 1)
                    def _():
                        fire_inputs(g + 2, st)

                    fire_scatters(st)

            drain_scatters(0)
            drain_scatters(1)
            plsc.subcore_barrier()
            orow = s * PPT
            pltpu.sync_copy(acc.at[pl.ds(PADR + orow, PPT), :],
                            out_hbm.at[pid, pl.ds(orow, PPT), :])
            plsc.subcore_barrier()

    return pl.kernel(
        body,
        out_type=jax.ShapeDtypeStruct((NC * KCH, HW, 8), jnp.float32),
        mesh=mesh,
        compiler_params=pltpu.CompilerParams(
            needs_layout_passes=False, use_tc_tiling_on_sc=False),
        scratch_types=[
            pltpu.VMEM_SHARED((R, 8), jnp.float32),
            pltpu.VMEM((512, 8), jnp.float32),
            pltpu.VMEM((2, G * 8), jnp.float32),
            pltpu.VMEM((2, 2 * G), jnp.float32),
            pltpu.VMEM((2, 2 * G), jnp.float32),
            pltpu.VMEM((2, 4, NSUB, 128), jnp.int32),
            pltpu.VMEM((2, 4, G, 8), jnp.float32),
            pltpu.SemaphoreType.DMA,
            pltpu.SemaphoreType.DMA,
            pltpu.SemaphoreType.DMA,
            pltpu.SemaphoreType.DMA,
            pltpu.SemaphoreType.DMA,
        ],
    )


def _p3_body(num_ref, nrm_ref, out_ref):
    nrm = nrm_ref[0, 0, :, 0:1]
    out_ref[...] = num_ref[...] / (nrm[None, None] + 1e-22)


def kernel(tenInput, tenFlow, tenMetric):
    N, C, H, W = tenInput.shape
    HW = H * W
    BP = 512
    BH = 8
    G = 384

    inp = tenInput.reshape(N, C, HW)
    met = tenMetric.reshape(N, 1, HW)

    v0, e = pl.pallas_call(
        _p1_body,
        grid=(N, HW // BP),
        in_specs=[
            pl.BlockSpec((1, C, BP), lambda n, j: (n, 0, j)),
            pl.BlockSpec((1, 1, BP), lambda n, j: (n, 0, j)),
        ],
        out_specs=[
            pl.BlockSpec((1, C, BP), lambda n, j: (n, 0, j)),
            pl.BlockSpec((1, 1, BP), lambda n, j: (n, 0, j)),
        ],
        out_shape=[
            jax.ShapeDtypeStruct((N, C, HW), jnp.float32),
            jax.ShapeDtypeStruct((N, 1, HW), jnp.float32),
        ],
    )(inp, met)

    idx, wpl = pl.pallas_call(
        functools.partial(_p2_body, H=H, W=W, BH=BH),
        grid=(N, H // BH),
        in_specs=[pl.BlockSpec((1, 2, BH, W), lambda n, i: (n, 0, i, 0))],
        out_specs=[
            pl.BlockSpec((1, 4, BH, W), lambda n, i: (n, 0, i, 0)),
            pl.BlockSpec((1, 4, BH, W), lambda n, i: (n, 0, i, 0)),
        ],
        out_shape=[
            jax.ShapeDtypeStruct((N, 4, H, W), jnp.int32),
            jax.ShapeDtypeStruct((N, 4, H, W), jnp.float32),
        ],
    )(tenFlow)

    e13 = jnp.concatenate([e, jnp.zeros((N, 7, HW), jnp.float32)], axis=1)
    vfull = jnp.concatenate([v0.reshape(N, 12, 8, HW), e13[:, None]], axis=1)
    v26 = vfull.transpose(0, 1, 3, 2).reshape(N * KCH, HW * 8)

    idx_r = idx.reshape(N, 4, HW // 128, 128)
    wpl_r = wpl.reshape(N, 2, 2, HW)
    zeros = jnp.zeros((512, 8), jnp.float32)

    sc = _make_sc_scatter(HW, G)
    out26 = sc(v26, idx_r, wpl_r, zeros)

    o4 = out26.reshape(N, KCH, HW, 8)
    d = pl.pallas_call(
        _p3_body,
        grid=(N, HW // BP),
        in_specs=[
            pl.BlockSpec((1, 12, BP, 8), lambda n, j: (n, 0, j, 0)),
            pl.BlockSpec((1, 1, BP, 8), lambda n, j: (n, 12, j, 0)),
        ],
        out_specs=pl.BlockSpec((1, 12, BP, 8), lambda n, j: (n, 0, j, 0)),
        out_shape=jax.ShapeDtypeStruct((N, 12, HW, 8), jnp.float32),
    )(o4, o4)

    return d.transpose(0, 1, 3, 2).reshape(N, C, H, W)


# speedup vs baseline: 2.9973x; 1.2772x over previous
import functools
import jax
import jax.numpy as jnp
from jax import lax
from jax.experimental import pallas as pl
from jax.experimental.pallas import tpu as pltpu
from jax.experimental.pallas import tpu_sc as plsc

NC = 2
NS = 16
PADR = 8
KCH = 13


def _p1_body(inp_ref, met_ref, v_ref):
    e = jnp.exp(met_ref[...])
    v_ref[:, 0:96] = inp_ref[...] * e
    v_ref[:, 96:97] = e
    v_ref[:, 97:104] = jnp.zeros_like(v_ref[:, 97:104])


def _p2_body(flow_ref, idx_ref, w_ref, *, H, W, BH):
    i = pl.program_id(1)
    fx = flow_ref[0, 0]
    fy = flow_ref[0, 1]
    yb = i * BH
    yi = lax.broadcasted_iota(jnp.int32, (BH, W), 0) + yb
    xi = lax.broadcasted_iota(jnp.int32, (BH, W), 1)
    yf = yi.astype(jnp.float32)
    xf = xi.astype(jnp.float32)
    Xp = xf + fx
    Yp = yf + fy
    x0 = jnp.floor(Xp)
    y0 = jnp.floor(Yp)
    x1 = x0 + 1.0
    y1 = y0 + 1.0
    w_nw = (x1 - Xp) * (y1 - Yp)
    w_ne = (Xp - x0) * (y1 - Yp)
    w_sw = (x1 - Xp) * (Yp - y0)
    w_se = (Xp - x0) * (Yp - y0)
    x0i = x0.astype(jnp.int32)
    y0i = y0.astype(jnp.int32)
    x1i = x0i + 1
    y1i = y0i + 1
    mx0 = ((x0i >= 0) & (x0i < W)).astype(jnp.float32)
    mx1 = ((x1i >= 0) & (x1i < W)).astype(jnp.float32)
    my0 = (y0i >= 0) & (y0i < H)
    my1 = (y1i >= 0) & (y1i < H)
    my0f = my0.astype(jnp.float32)
    my1f = my1.astype(jnp.float32)

    p = yi * W + xi
    dump = (p & 3) + 1

    act_n = my0 & (x0i >= -1) & (x0i <= W - 1)
    act_s = my1 & (x0i >= -1) & (x0i <= W - 1)
    idx_n = jnp.where(act_n, y0i * W + x0i + PADR, dump)
    idx_s = jnp.where(act_s, y1i * W + x0i + PADR, dump)

    idx_ref[0, 0] = idx_n
    idx_ref[0, 1] = idx_n + 1
    idx_ref[0, 2] = idx_s
    idx_ref[0, 3] = idx_s + 1
    w_ref[0, 0] = w_nw * (my0f * mx0)
    w_ref[0, 1] = w_ne * (my0f * mx1)
    w_ref[0, 2] = w_sw * (my1f * mx0)
    w_ref[0, 3] = w_se * (my1f * mx1)


def _make_sc_scatter(HW, G):
    R = HW + 2 * PADR
    PPT = HW // NS
    NGRP = PPT // G
    NSUB = G // 128
    ZR = 512
    ZN = (R // NS) // ZR
    ZREM = R // NS - ZN * ZR
    mesh = plsc.VectorSubcoreMesh(core_axis_name="c", subcore_axis_name="s")

    def body(v_hbm, idx_hbm, w_hbm, z_hbm, out_hbm,
             acc, zbuf, vbuf, wbn, wbs, ibuf, sb,
             sem_in0, sem_in1, sem_sc0, sem_sc1, sem_z):
        c = lax.axis_index("c")
        s = lax.axis_index("s")
        sem_in = (sem_in0, sem_in1)
        sem_sc = (sem_sc0, sem_sc1)
        pltpu.sync_copy(z_hbm, zbuf)

        lanes = lax.iota(jnp.int32, 16)
        wpat = lanes >> 3
        cpat = lanes & 7
        base_p = s * PPT
        ib128 = s * (PPT // 128)

        @pl.loop(0, KCH)
        def _pass(k):
            pid = c * KCH + k

            z0 = s * (R // NS)
            for j in range(ZN):
                pltpu.async_copy(zbuf, acc.at[pl.ds(z0 + j * ZR, ZR), :], sem_z)
            if ZREM:
                pltpu.async_copy(zbuf.at[pl.ds(0, ZREM), :],
                                 acc.at[pl.ds(z0 + ZN * ZR, ZREM), :], sem_z)
            for j in range(ZN):
                pltpu.make_async_copy(zbuf, acc.at[pl.ds(z0 + j * ZR, ZR), :], sem_z).wait()
            if ZREM:
                pltpu.make_async_copy(zbuf.at[pl.ds(0, ZREM), :],
                                      acc.at[pl.ds(z0 + ZN * ZR, ZREM), :], sem_z).wait()
            plsc.subcore_barrier()

            def in_copies(g, st):
                p0 = base_p + g * G
                return (
                    (v_hbm.at[pid, :, pl.ds(p0, G)], vbuf.at[st]),
                    (w_hbm.at[c, 0, 0, pl.ds(p0, G)], wbn.at[st, pl.ds(0, G)]),
                    (w_hbm.at[c, 0, 1, pl.ds(p0, G)], wbn.at[st, pl.ds(G, G)]),
                    (w_hbm.at[c, 1, 0, pl.ds(p0, G)], wbs.at[st, pl.ds(0, G)]),
                    (w_hbm.at[c, 1, 1, pl.ds(p0, G)], wbs.at[st, pl.ds(G, G)]),
                )

            def fire_inputs(g, st):
                for src, dst in in_copies(g, st):
                    pltpu.async_copy(src, dst, sem_in[st])

            def wait_inputs(g, st):
                for src, dst in in_copies(g, st):
                    pltpu.make_async_copy(src, dst, sem_in[st]).wait()

            def sc_copies(st):
                return tuple(
                    (sb.at[st, cell, pl.ds(j * 128, 128), :],
                     acc.at[ibuf.at[st, cell, j]])
                    for cell in range(4) for j in range(NSUB))

            def fire_scatters(st):
                for src, dst in sc_copies(st):
                    pltpu.async_copy(src, dst, sem_sc[st], add=True)

            def drain_scatters(st):
                for src, dst in sc_copies(st):
                    pltpu.make_async_copy(src, dst, sem_sc[st]).wait()

            fire_inputs(0, 0)
            fire_inputs(1, 1)

            @pl.loop(0, NGRP // 2)
            def _gg(gg):
                for st in (0, 1):
                    g = gg * 2 + st

                    @pl.when(gg > 0)
                    def _():
                        drain_scatters(st)

                    pltpu.sync_copy(
                        idx_hbm.at[c, :, pl.ds(ib128 + g * NSUB, NSUB), :],
                        ibuf.at[st])
                    wait_inputs(g, st)

                    @pl.loop(0, G // 2)
                    def _pair(q):
                        q2 = q * 2
                        v2 = plsc.load_gather(vbuf.at[st], [cpat, q2 + wpat])
                        wi0 = q2 + wpat
                        wn0 = plsc.load_gather(wbn.at[st], [wi0])
                        wn1 = plsc.load_gather(wbn.at[st], [wi0 + G])
                        ws0 = plsc.load_gather(wbs.at[st], [wi0])
                        ws1 = plsc.load_gather(wbs.at[st], [wi0 + G])
                        ridx = q2 + wpat
                        plsc.store_scatter(sb.at[st, 0], [ridx, cpat], v2 * wn0)
                        plsc.store_scatter(sb.at[st, 1], [ridx, cpat], v2 * wn1)
                        plsc.store_scatter(sb.at[st, 2], [ridx, cpat], v2 * ws0)
                        plsc.store_scatter(sb.at[st, 3], [ridx, cpat], v2 * ws1)

                    @pl.when(gg < NGRP // 2 - 1)
                    def _():
                        fire_inputs(g + 2, st)

                    fire_scatters(st)

            drain_scatters(0)
            drain_scatters(1)
            plsc.subcore_barrier()
            orow = s * PPT
            pltpu.sync_copy(acc.at[pl.ds(PADR + orow, PPT), :],
                            out_hbm.at[pid, pl.ds(orow, PPT), :])
            plsc.subcore_barrier()

    return pl.kernel(
        body,
        out_type=jax.ShapeDtypeStruct((NC * KCH, HW, 8), jnp.float32),
        mesh=mesh,
        compiler_params=pltpu.CompilerParams(
            needs_layout_passes=False, use_tc_tiling_on_sc=False),
        scratch_types=[
            pltpu.VMEM_SHARED((R, 8), jnp.float32),
            pltpu.VMEM((512, 8), jnp.float32),
            pltpu.VMEM((2, 8, G), jnp.float32),
            pltpu.VMEM((2, 2 * G), jnp.float32),
            pltpu.VMEM((2, 2 * G), jnp.float32),
            pltpu.VMEM((2, 4, NSUB, 128), jnp.int32),
            pltpu.VMEM((2, 4, G, 8), jnp.float32),
            pltpu.SemaphoreType.DMA,
            pltpu.SemaphoreType.DMA,
            pltpu.SemaphoreType.DMA,
            pltpu.SemaphoreType.DMA,
            pltpu.SemaphoreType.DMA,
        ],
    )


def _p3_body(num_ref, nrm_ref, out_ref):
    nrm = nrm_ref[0, 0, :, 0:1]
    out_ref[...] = num_ref[...] / (nrm[None, None] + 1e-22)


def kernel(tenInput, tenFlow, tenMetric):
    N, C, H, W = tenInput.shape
    HW = H * W
    BP = 512
    BH = 8
    G = 384

    inp = tenInput.reshape(N, C, HW)
    met = tenMetric.reshape(N, 1, HW)

    vcm = pl.pallas_call(
        _p1_body,
        grid=(N, HW // BP),
        in_specs=[
            pl.BlockSpec((1, C, BP), lambda n, j: (n, 0, j)),
            pl.BlockSpec((1, 1, BP), lambda n, j: (n, 0, j)),
        ],
        out_specs=pl.BlockSpec((1, 104, BP), lambda n, j: (n, 0, j)),
        out_shape=jax.ShapeDtypeStruct((N, 104, HW), jnp.float32),
    )(inp, met)

    idx, wpl = pl.pallas_call(
        functools.partial(_p2_body, H=H, W=W, BH=BH),
        grid=(N, H // BH),
        in_specs=[pl.BlockSpec((1, 2, BH, W), lambda n, i: (n, 0, i, 0))],
        out_specs=[
            pl.BlockSpec((1, 4, BH, W), lambda n, i: (n, 0, i, 0)),
            pl.BlockSpec((1, 4, BH, W), lambda n, i: (n, 0, i, 0)),
        ],
        out_shape=[
            jax.ShapeDtypeStruct((N, 4, H, W), jnp.int32),
            jax.ShapeDtypeStruct((N, 4, H, W), jnp.float32),
        ],
    )(tenFlow)

    v26 = vcm.reshape(N * KCH, 8, HW)

    idx_r = idx.reshape(N, 4, HW // 128, 128)
    wpl_r = wpl.reshape(N, 2, 2, HW)
    zeros = jnp.zeros((512, 8), jnp.float32)

    sc = _make_sc_scatter(HW, G)
    out26 = sc(v26, idx_r, wpl_r, zeros)

    o4 = out26.reshape(N, KCH, HW, 8)
    d = pl.pallas_call(
        _p3_body,
        grid=(N, HW // BP),
        in_specs=[
            pl.BlockSpec((1, 12, BP, 8), lambda n, j: (n, 0, j, 0)),
            pl.BlockSpec((1, 1, BP, 8), lambda n, j: (n, 12, j, 0)),
        ],
        out_specs=pl.BlockSpec((1, 12, BP, 8), lambda n, j: (n, 0, j, 0)),
        out_shape=jax.ShapeDtypeStruct((N, 12, HW, 8), jnp.float32),
    )(o4, o4)

    return d.transpose(0, 1, 3, 2).reshape(N, C, H, W)


# speedup vs baseline: 3.3168x; 1.1066x over previous
import functools
import jax
import jax.numpy as jnp
from jax import lax
from jax.experimental import pallas as pl
from jax.experimental.pallas import tpu as pltpu
from jax.experimental.pallas import tpu_sc as plsc

NC = 2
NS = 16
PADR = 8
KCH = 13


def _p2_body(flow_ref, met_ref, idx_ref, w_ref, *, H, W, BH):
    i = pl.program_id(1)
    fx = flow_ref[0, 0]
    fy = flow_ref[0, 1]
    e = jnp.exp(met_ref[0, 0])
    yb = i * BH
    yi = lax.broadcasted_iota(jnp.int32, (BH, W), 0) + yb
    xi = lax.broadcasted_iota(jnp.int32, (BH, W), 1)
    yf = yi.astype(jnp.float32)
    xf = xi.astype(jnp.float32)
    Xp = xf + fx
    Yp = yf + fy
    x0 = jnp.floor(Xp)
    y0 = jnp.floor(Yp)
    x1 = x0 + 1.0
    y1 = y0 + 1.0
    w_nw = (x1 - Xp) * (y1 - Yp)
    w_ne = (Xp - x0) * (y1 - Yp)
    w_sw = (x1 - Xp) * (Yp - y0)
    w_se = (Xp - x0) * (Yp - y0)
    x0i = x0.astype(jnp.int32)
    y0i = y0.astype(jnp.int32)
    x1i = x0i + 1
    y1i = y0i + 1
    mx0 = ((x0i >= 0) & (x0i < W)).astype(jnp.float32)
    mx1 = ((x1i >= 0) & (x1i < W)).astype(jnp.float32)
    my0 = (y0i >= 0) & (y0i < H)
    my1 = (y1i >= 0) & (y1i < H)
    my0f = my0.astype(jnp.float32)
    my1f = my1.astype(jnp.float32)

    p = yi * W + xi
    dump = (p & 3) + 1

    act_n = my0 & (x0i >= -1) & (x0i <= W - 1)
    act_s = my1 & (x0i >= -1) & (x0i <= W - 1)
    idx_n = jnp.where(act_n, y0i * W + x0i + PADR, dump)
    idx_s = jnp.where(act_s, y1i * W + x0i + PADR, dump)

    idx_ref[0, 0] = idx_n
    idx_ref[0, 1] = idx_n + 1
    idx_ref[0, 2] = idx_s
    idx_ref[0, 3] = idx_s + 1
    w_ref[0, 0] = w_nw * (my0f * mx0) * e
    w_ref[0, 1] = w_ne * (my0f * mx1) * e
    w_ref[0, 2] = w_sw * (my1f * mx0) * e
    w_ref[0, 3] = w_se * (my1f * mx1) * e


def _make_sc_scatter(HW, G):
    R = HW + 2 * PADR
    PPT = HW // NS
    NGRP = PPT // G
    NSUB = G // 128
    ZR = 512
    ZN = (R // NS) // ZR
    ZREM = R // NS - ZN * ZR
    mesh = plsc.VectorSubcoreMesh(core_axis_name="c", subcore_axis_name="s")

    def body(v_hbm, vn_hbm, idx_hbm, w_hbm, z_hbm, out_hbm,
             acc, zbuf, vbuf, wbn, wbs, ibuf, sb,
             sem_in0, sem_in1, sem_sc0, sem_sc1, sem_z):
        c = lax.axis_index("c")
        s = lax.axis_index("s")
        sem_in = (sem_in0, sem_in1)
        sem_sc = (sem_sc0, sem_sc1)
        pltpu.sync_copy(z_hbm, zbuf)

        lanes = lax.iota(jnp.int32, 16)
        wpat = lanes >> 3
        cpat = lanes & 7
        base_p = s * PPT
        ib128 = s * (PPT // 128)

        @pl.loop(0, KCH)
        def _pass(k):
            pid = c * KCH + k

            z0 = s * (R // NS)
            for j in range(ZN):
                pltpu.async_copy(zbuf, acc.at[pl.ds(z0 + j * ZR, ZR), :], sem_z)
            if ZREM:
                pltpu.async_copy(zbuf.at[pl.ds(0, ZREM), :],
                                 acc.at[pl.ds(z0 + ZN * ZR, ZREM), :], sem_z)
            for j in range(ZN):
                pltpu.make_async_copy(zbuf, acc.at[pl.ds(z0 + j * ZR, ZR), :], sem_z).wait()
            if ZREM:
                pltpu.make_async_copy(zbuf.at[pl.ds(0, ZREM), :],
                                      acc.at[pl.ds(z0 + ZN * ZR, ZREM), :], sem_z).wait()
            plsc.subcore_barrier()

            def v_copy(g, st):
                p0 = base_p + g * G
                vpid = c * 12 + jnp.minimum(k, 11)
                return (v_hbm.at[vpid, :, pl.ds(p0, G)], vbuf.at[st])

            def in_copies(g, st):
                p0 = base_p + g * G
                return (
                    (w_hbm.at[c, 0, 0, pl.ds(p0, G)], wbn.at[st, pl.ds(0, G)]),
                    (w_hbm.at[c, 0, 1, pl.ds(p0, G)], wbn.at[st, pl.ds(G, G)]),
                    (w_hbm.at[c, 1, 0, pl.ds(p0, G)], wbs.at[st, pl.ds(0, G)]),
                    (w_hbm.at[c, 1, 1, pl.ds(p0, G)], wbs.at[st, pl.ds(G, G)]),
                )

            def fire_inputs(g, st):
                @pl.when(k < 12)
                def _():
                    src, dst = v_copy(g, st)
                    pltpu.async_copy(src, dst, sem_in[st])

                @pl.when(k == 12)
                def _():
                    p0 = base_p + g * G
                    pltpu.async_copy(vn_hbm.at[:, pl.ds(p0, G)],
                                     vbuf.at[st], sem_in[st])

                for src, dst in in_copies(g, st):
                    pltpu.async_copy(src, dst, sem_in[st])

            def wait_inputs(g, st):
                src, dst = v_copy(g, st)
                pltpu.make_async_copy(src, dst, sem_in[st]).wait()
                for src, dst in in_copies(g, st):
                    pltpu.make_async_copy(src, dst, sem_in[st]).wait()

            def sc_copies(st):
                return tuple(
                    (sb.at[st, cell, pl.ds(j * 128, 128), :],
                     acc.at[ibuf.at[st, cell, j]])
                    for cell in range(4) for j in range(NSUB))

            def fire_scatters(st):
                for src, dst in sc_copies(st):
                    pltpu.async_copy(src, dst, sem_sc[st], add=True)

            def drain_scatters(st):
                for src, dst in sc_copies(st):
                    pltpu.make_async_copy(src, dst, sem_sc[st]).wait()

            fire_inputs(0, 0)
            fire_inputs(1, 1)

            @pl.loop(0, NGRP // 2)
            def _gg(gg):
                for st in (0, 1):
                    g = gg * 2 + st

                    @pl.when(gg > 0)
                    def _():
                        drain_scatters(st)

                    pltpu.sync_copy(
                        idx_hbm.at[c, :, pl.ds(ib128 + g * NSUB, NSUB), :],
                        ibuf.at[st])
                    wait_inputs(g, st)

                    @pl.loop(0, G // 2, unroll=4)
                    def _pair(q):
                        q2 = q * 2
                        wi0 = q2 + wpat
                        v2 = plsc.load_gather(vbuf.at[st], [cpat, wi0])
                        wn0 = plsc.load_gather(wbn.at[st], [wi0])
                        wn1 = plsc.load_gather(wbn.at[st], [wi0 + G])
                        ws0 = plsc.load_gather(wbs.at[st], [wi0])
                        ws1 = plsc.load_gather(wbs.at[st], [wi0 + G])
                        plsc.store_scatter(sb.at[st, 0], [wi0, cpat], v2 * wn0)
                        plsc.store_scatter(sb.at[st, 1], [wi0, cpat], v2 * wn1)
                        plsc.store_scatter(sb.at[st, 2], [wi0, cpat], v2 * ws0)
                        plsc.store_scatter(sb.at[st, 3], [wi0, cpat], v2 * ws1)

                    @pl.when(gg < NGRP // 2 - 1)
                    def _():
                        fire_inputs(g + 2, st)

                    fire_scatters(st)

            drain_scatters(0)
            drain_scatters(1)
            plsc.subcore_barrier()
            orow = s * PPT
            pltpu.sync_copy(acc.at[pl.ds(PADR + orow, PPT), :],
                            out_hbm.at[pid, pl.ds(orow, PPT), :])
            plsc.subcore_barrier()

    return pl.kernel(
        body,
        out_type=jax.ShapeDtypeStruct((NC * KCH, HW, 8), jnp.float32),
        mesh=mesh,
        compiler_params=pltpu.CompilerParams(
            needs_layout_passes=False, use_tc_tiling_on_sc=False),
        scratch_types=[
            pltpu.VMEM_SHARED((R, 8), jnp.float32),
            pltpu.VMEM((512, 8), jnp.float32),
            pltpu.VMEM((2, 8, G), jnp.float32),
            pltpu.VMEM((2, 2 * G), jnp.float32),
            pltpu.VMEM((2, 2 * G), jnp.float32),
            pltpu.VMEM((2, 4, NSUB, 128), jnp.int32),
            pltpu.VMEM((2, 4, G, 8), jnp.float32),
            pltpu.SemaphoreType.DMA,
            pltpu.SemaphoreType.DMA,
            pltpu.SemaphoreType.DMA,
            pltpu.SemaphoreType.DMA,
            pltpu.SemaphoreType.DMA,
        ],
    )


def _p3_body(num_ref, nrm_ref, out_ref):
    nrm = nrm_ref[0, 0, :, 0:1]
    out_ref[...] = num_ref[...] / (nrm[None, None] + 1e-22)


def kernel(tenInput, tenFlow, tenMetric):
    N, C, H, W = tenInput.shape
    HW = H * W
    BP = 512
    BH = 8
    G = 384

    idx, wpl = pl.pallas_call(
        functools.partial(_p2_body, H=H, W=W, BH=BH),
        grid=(N, H // BH),
        in_specs=[
            pl.BlockSpec((1, 2, BH, W), lambda n, i: (n, 0, i, 0)),
            pl.BlockSpec((1, 1, BH, W), lambda n, i: (n, 0, i, 0)),
        ],
        out_specs=[
            pl.BlockSpec((1, 4, BH, W), lambda n, i: (n, 0, i, 0)),
            pl.BlockSpec((1, 4, BH, W), lambda n, i: (n, 0, i, 0)),
        ],
        out_shape=[
            jax.ShapeDtypeStruct((N, 4, H, W), jnp.int32),
            jax.ShapeDtypeStruct((N, 4, H, W), jnp.float32),
        ],
    )(tenFlow, tenMetric)

    v24 = tenInput.reshape(N * 12, 8, HW)
    vnorm = jnp.zeros((8, HW), jnp.float32).at[0, :].set(1.0)

    idx_r = idx.reshape(N, 4, HW // 128, 128)
    wpl_r = wpl.reshape(N, 2, 2, HW)
    zeros = jnp.zeros((512, 8), jnp.float32)

    sc = _make_sc_scatter(HW, G)
    out26 = sc(v24, vnorm, idx_r, wpl_r, zeros)

    o4 = out26.reshape(N, KCH, HW, 8)
    d = pl.pallas_call(
        _p3_body,
        grid=(N, HW // BP),
        in_specs=[
            pl.BlockSpec((1, 12, BP, 8), lambda n, j: (n, 0, j, 0)),
            pl.BlockSpec((1, 1, BP, 8), lambda n, j: (n, 12, j, 0)),
        ],
        out_specs=pl.BlockSpec((1, 12, BP, 8), lambda n, j: (n, 0, j, 0)),
        out_shape=jax.ShapeDtypeStruct((N, 12, HW, 8), jnp.float32),
    )(o4, o4)

    return d.transpose(0, 1, 3, 2).reshape(N, C, H, W)


# speedup vs baseline: 3.3471x; 1.0092x over previous
import functools
import jax
import jax.numpy as jnp
from jax import lax
from jax.experimental import pallas as pl
from jax.experimental.pallas import tpu as pltpu
from jax.experimental.pallas import tpu_sc as plsc

NC = 2
NS = 16
PADR = 8
KCH = 13


def _p2_body(flow_ref, met_ref, idx_ref, w_ref, *, H, W, BH):
    i = pl.program_id(1)
    fx = flow_ref[0, 0]
    fy = flow_ref[0, 1]
    e = jnp.exp(met_ref[0, 0])
    yb = i * BH
    yi = lax.broadcasted_iota(jnp.int32, (BH, W), 0) + yb
    xi = lax.broadcasted_iota(jnp.int32, (BH, W), 1)
    yf = yi.astype(jnp.float32)
    xf = xi.astype(jnp.float32)
    Xp = xf + fx
    Yp = yf + fy
    x0 = jnp.floor(Xp)
    y0 = jnp.floor(Yp)
    x1 = x0 + 1.0
    y1 = y0 + 1.0
    w_nw = (x1 - Xp) * (y1 - Yp)
    w_ne = (Xp - x0) * (y1 - Yp)
    w_sw = (x1 - Xp) * (Yp - y0)
    w_se = (Xp - x0) * (Yp - y0)
    x0i = x0.astype(jnp.int32)
    y0i = y0.astype(jnp.int32)
    x1i = x0i + 1
    y1i = y0i + 1
    mx0 = ((x0i >= 0) & (x0i < W)).astype(jnp.float32)
    mx1 = ((x1i >= 0) & (x1i < W)).astype(jnp.float32)
    my0 = (y0i >= 0) & (y0i < H)
    my1 = (y1i >= 0) & (y1i < H)
    my0f = my0.astype(jnp.float32)
    my1f = my1.astype(jnp.float32)

    p = yi * W + xi
    dump = (p & 3) + 1

    act_n = my0 & (x0i >= -1) & (x0i <= W - 1)
    act_s = my1 & (x0i >= -1) & (x0i <= W - 1)
    idx_n = jnp.where(act_n, y0i * W + x0i + PADR, dump)
    idx_s = jnp.where(act_s, y1i * W + x0i + PADR, dump)

    idx_ref[0, 0] = idx_n
    idx_ref[0, 1] = idx_n + 1
    idx_ref[0, 2] = idx_s
    idx_ref[0, 3] = idx_s + 1
    w_ref[0, 0] = w_nw * (my0f * mx0) * e
    w_ref[0, 1] = w_ne * (my0f * mx1) * e
    w_ref[0, 2] = w_sw * (my1f * mx0) * e
    w_ref[0, 3] = w_se * (my1f * mx1) * e


def _make_sc_scatter(HW, G):
    R = HW + 2 * PADR
    PPT = HW // NS
    NGRP = PPT // G
    NSUB = G // 128
    ZR = 256
    ZN = (R // NS) // ZR
    ZREM = R // NS - ZN * ZR
    mesh = plsc.VectorSubcoreMesh(core_axis_name="c", subcore_axis_name="s")

    def body(v_hbm, vn_hbm, idx_hbm, w_hbm, z_hbm, out_hbm,
             acc, zbuf, vbuf, wbn, wbs, ibuf, sb,
             sem_in0, sem_in1, sem_sc0, sem_sc1, sem_z):
        c = lax.axis_index("c")
        s = lax.axis_index("s")
        sem_in = (sem_in0, sem_in1)
        sem_sc = (sem_sc0, sem_sc1)
        pltpu.sync_copy(z_hbm, zbuf)

        lanes = lax.iota(jnp.int32, 16)
        wpat = lanes >> 3
        cpat = lanes & 7
        base_p = s * PPT
        ib128 = s * (PPT // 128)

        @pl.loop(0, KCH)
        def _pass(k):
            pid = c * KCH + k

            z0 = s * (R // NS)
            for j in range(ZN):
                pltpu.async_copy(zbuf, acc.at[pl.ds(z0 + j * ZR, ZR), :], sem_z)
            if ZREM:
                pltpu.async_copy(zbuf.at[pl.ds(0, ZREM), :],
                                 acc.at[pl.ds(z0 + ZN * ZR, ZREM), :], sem_z)
            for j in range(ZN):
                pltpu.make_async_copy(zbuf, acc.at[pl.ds(z0 + j * ZR, ZR), :], sem_z).wait()
            if ZREM:
                pltpu.make_async_copy(zbuf.at[pl.ds(0, ZREM), :],
                                      acc.at[pl.ds(z0 + ZN * ZR, ZREM), :], sem_z).wait()
            plsc.subcore_barrier()

            def v_copy(g, st):
                p0 = base_p + g * G
                vpid = c * 12 + jnp.minimum(k, 11)
                return (v_hbm.at[vpid, :, pl.ds(p0, G)], vbuf.at[st])

            def in_copies(g, st):
                p0 = base_p + g * G
                return (
                    (w_hbm.at[c, 0, 0, pl.ds(p0, G)], wbn.at[st, pl.ds(0, G)]),
                    (w_hbm.at[c, 0, 1, pl.ds(p0, G)], wbn.at[st, pl.ds(G, G)]),
                    (w_hbm.at[c, 1, 0, pl.ds(p0, G)], wbs.at[st, pl.ds(0, G)]),
                    (w_hbm.at[c, 1, 1, pl.ds(p0, G)], wbs.at[st, pl.ds(G, G)]),
                )

            def fire_inputs(g, st):
                @pl.when(k < 12)
                def _():
                    src, dst = v_copy(g, st)
                    pltpu.async_copy(src, dst, sem_in[st])

                @pl.when(k == 12)
                def _():
                    p0 = base_p + g * G
                    pltpu.async_copy(vn_hbm.at[:, pl.ds(p0, G)],
                                     vbuf.at[st], sem_in[st])

                for src, dst in in_copies(g, st):
                    pltpu.async_copy(src, dst, sem_in[st])

            def wait_inputs(g, st):
                src, dst = v_copy(g, st)
                pltpu.make_async_copy(src, dst, sem_in[st]).wait()
                for src, dst in in_copies(g, st):
                    pltpu.make_async_copy(src, dst, sem_in[st]).wait()

            def sc_copies(st):
                return tuple(
                    (sb.at[st, cell, pl.ds(j * 128, 128), :],
                     acc.at[ibuf.at[st, cell, j]])
                    for cell in range(4) for j in range(NSUB))

            def fire_scatters(st):
                for src, dst in sc_copies(st):
                    pltpu.async_copy(src, dst, sem_sc[st], add=True)

            def drain_scatters(st):
                for src, dst in sc_copies(st):
                    pltpu.make_async_copy(src, dst, sem_sc[st]).wait()

            fire_inputs(0, 0)
            fire_inputs(1, 1)

            @pl.loop(0, NGRP // 2)
            def _gg(gg):
                for st in (0, 1):
                    g = gg * 2 + st

                    @pl.when(gg > 0)
                    def _():
                        drain_scatters(st)

                    pltpu.sync_copy(
                        idx_hbm.at[c, :, pl.ds(ib128 + g * NSUB, NSUB), :],
                        ibuf.at[st])
                    wait_inputs(g, st)

                    @pl.loop(0, G // 2, unroll=4)
                    def _pair(q):
                        q2 = q * 2
                        wi0 = q2 + wpat
                        v2 = plsc.load_gather(vbuf.at[st], [cpat, wi0])
                        wn0 = plsc.load_gather(wbn.at[st], [wi0])
                        wn1 = plsc.load_gather(wbn.at[st], [wi0 + G])
                        ws0 = plsc.load_gather(wbs.at[st], [wi0])
                        ws1 = plsc.load_gather(wbs.at[st], [wi0 + G])
                        plsc.store_scatter(sb.at[st, 0], [wi0, cpat], v2 * wn0)
                        plsc.store_scatter(sb.at[st, 1], [wi0, cpat], v2 * wn1)
                        plsc.store_scatter(sb.at[st, 2], [wi0, cpat], v2 * ws0)
                        plsc.store_scatter(sb.at[st, 3], [wi0, cpat], v2 * ws1)

                    @pl.when(gg < NGRP // 2 - 1)
                    def _():
                        fire_inputs(g + 2, st)

                    fire_scatters(st)

            drain_scatters(0)
            drain_scatters(1)
            plsc.subcore_barrier()
            orow = s * PPT
            pltpu.sync_copy(acc.at[pl.ds(PADR + orow, PPT), :],
                            out_hbm.at[pid, pl.ds(orow, PPT), :])
            plsc.subcore_barrier()

    return pl.kernel(
        body,
        out_type=jax.ShapeDtypeStruct((NC * KCH, HW, 8), jnp.float32),
        mesh=mesh,
        compiler_params=pltpu.CompilerParams(
            needs_layout_passes=False, use_tc_tiling_on_sc=False),
        scratch_types=[
            pltpu.VMEM_SHARED((R, 8), jnp.float32),
            pltpu.VMEM((256, 8), jnp.float32),
            pltpu.VMEM((2, 8, G), jnp.float32),
            pltpu.VMEM((2, 2 * G), jnp.float32),
            pltpu.VMEM((2, 2 * G), jnp.float32),
            pltpu.VMEM((2, 4, NSUB, 128), jnp.int32),
            pltpu.VMEM((2, 4, G, 8), jnp.float32),
            pltpu.SemaphoreType.DMA,
            pltpu.SemaphoreType.DMA,
            pltpu.SemaphoreType.DMA,
            pltpu.SemaphoreType.DMA,
            pltpu.SemaphoreType.DMA,
        ],
    )


def _p3_body(num_ref, nrm_ref, out_ref):
    nrm = nrm_ref[0, 0, :, 0:1]
    out_ref[...] = num_ref[...] / (nrm[None, None] + 1e-22)


def kernel(tenInput, tenFlow, tenMetric):
    N, C, H, W = tenInput.shape
    HW = H * W
    BP = 512
    BH = 8
    G = 512

    idx, wpl = pl.pallas_call(
        functools.partial(_p2_body, H=H, W=W, BH=BH),
        grid=(N, H // BH),
        in_specs=[
            pl.BlockSpec((1, 2, BH, W), lambda n, i: (n, 0, i, 0)),
            pl.BlockSpec((1, 1, BH, W), lambda n, i: (n, 0, i, 0)),
        ],
        out_specs=[
            pl.BlockSpec((1, 4, BH, W), lambda n, i: (n, 0, i, 0)),
            pl.BlockSpec((1, 4, BH, W), lambda n, i: (n, 0, i, 0)),
        ],
        out_shape=[
            jax.ShapeDtypeStruct((N, 4, H, W), jnp.int32),
            jax.ShapeDtypeStruct((N, 4, H, W), jnp.float32),
        ],
    )(tenFlow, tenMetric)

    v24 = tenInput.reshape(N * 12, 8, HW)
    vnorm = jnp.zeros((8, HW), jnp.float32).at[0, :].set(1.0)

    idx_r = idx.reshape(N, 4, HW // 128, 128)
    wpl_r = wpl.reshape(N, 2, 2, HW)
    zeros = jnp.zeros((256, 8), jnp.float32)

    sc = _make_sc_scatter(HW, G)
    out26 = sc(v24, vnorm, idx_r, wpl_r, zeros)

    o4 = out26.reshape(N, KCH, HW, 8)
    d = pl.pallas_call(
        _p3_body,
        grid=(N, HW // BP),
        in_specs=[
            pl.BlockSpec((1, 12, BP, 8), lambda n, j: (n, 0, j, 0)),
            pl.BlockSpec((1, 1, BP, 8), lambda n, j: (n, 12, j, 0)),
        ],
        out_specs=pl.BlockSpec((1, 12, BP, 8), lambda n, j: (n, 0, j, 0)),
        out_shape=jax.ShapeDtypeStruct((N, 12, HW, 8), jnp.float32),
    )(o4, o4)

    return d.transpose(0, 1, 3, 2).reshape(N, C, H, W)
